# baseline (speedup 1.0000x reference)
import functools

import jax
import jax.numpy as jnp
from jax import lax
from jax.experimental import pallas as pl
from jax.experimental.pallas import tpu as pltpu
from jax.experimental.pallas import tpu_sc as plsc

BL = 2048

NC, NS, LANES = 2, 16, 16
NW = NC * NS
CHE = 16 * 1024


def _tc_add(x_ref, w_ref, o_ref):
    o_ref[...] = x_ref[...] + w_ref[...][None]


def _tc_call(x, w):
    B, L, D = x.shape
    return pl.pallas_call(
        _tc_add,
        grid=(L // BL, B),
        in_specs=[
            pl.BlockSpec((1, BL, D), lambda l, b: (b, l, 0)),
            pl.BlockSpec((BL, D), lambda l, b: (l, 0)),
        ],
        out_specs=pl.BlockSpec((1, BL, D), lambda l, b: (b, l, 0)),
        out_shape=jax.ShapeDtypeStruct((B, L, D), x.dtype),
    )(x, w)


def _sc_add(x_hbm, w_hbm, out_hbm, bufx, bufw, sinx, sinw, sout):
    E = x_hbm.shape[0]
    Ew = w_hbm.shape[0]
    e_per_w = E // NW
    nchunk = e_per_w // CHE
    wid = lax.axis_index("s") * NC + lax.axis_index("c")
    base = wid * e_per_w
    wbase = lax.rem(base, Ew)

    def start_in(p, c):
        o = c * CHE
        pltpu.async_copy(x_hbm.at[pl.ds(base + o, CHE)], bufx.at[p], sinx[p])
        pltpu.async_copy(w_hbm.at[pl.ds(wbase + o, CHE)], bufw.at[p], sinw[p])

    def wait_in(p, c):
        o = c * CHE
        pltpu.make_async_copy(x_hbm.at[pl.ds(base + o, CHE)], bufx.at[p], sinx[p]).wait()
        pltpu.make_async_copy(w_hbm.at[pl.ds(wbase + o, CHE)], bufw.at[p], sinw[p]).wait()

    def start_out(p, c):
        o = c * CHE
        pltpu.async_copy(bufw.at[p], out_hbm.at[pl.ds(base + o, CHE)], sout[p])

    def wait_out(p, c):
        o = c * CHE
        pltpu.make_async_copy(bufw.at[p], out_hbm.at[pl.ds(base + o, CHE)], sout[p]).wait()

    start_in(0, 0)
    for c in range(nchunk):
        p = c % 2
        if c + 1 < nchunk:
            if c >= 1:
                wait_out(1 - p, c - 1)
            start_in(1 - p, c + 1)
        wait_in(p, c)

        @plsc.parallel_loop(0, CHE // LANES, 1, unroll=8)
        def add_body(i):
            plsc.addupdate(
                bufw.at[p].at[pl.ds(i * LANES, LANES)],
                bufx[p, pl.ds(i * LANES, LANES)],
            )

        start_out(p, c)
    wait_out(nchunk % 2, nchunk - 2)
    wait_out(1 - nchunk % 2, nchunk - 1)


def _sc_call(x_flat, w_flat):
    mesh = plsc.VectorSubcoreMesh(core_axis_name="c", subcore_axis_name="s")
    return functools.partial(
        pl.kernel,
        mesh=mesh,
        out_type=jax.ShapeDtypeStruct(x_flat.shape, jnp.float32),
        scratch_types=[
            pltpu.VMEM((2, CHE), jnp.float32),
            pltpu.VMEM((2, CHE), jnp.float32),
            [pltpu.SemaphoreType.DMA] * 2,
            [pltpu.SemaphoreType.DMA] * 2,
            [pltpu.SemaphoreType.DMA] * 2,
        ],
    )(_sc_add)(x_flat, w_flat)


def kernel(x, embed_weight):
    B, L, D = x.shape
    Bt = B - 1
    out_tc = _tc_call(x[:Bt], embed_weight)
    out_sc = _sc_call(x[Bt:].reshape(-1), embed_weight.reshape(-1))
    return jnp.concatenate([out_tc.reshape(-1), out_sc]).reshape(B, L, D)


# speedup vs baseline: 6.0390x; 6.0390x over previous
import jax
import jax.numpy as jnp
from jax.experimental import pallas as pl

BL = 512


def _add_kernel(x_ref, w_ref, o_ref):
    o_ref[...] = x_ref[...] + w_ref[...][None]


def kernel(x, embed_weight):
    B, L, D = x.shape
    return pl.pallas_call(
        _add_kernel,
        grid=(L // BL,),
        in_specs=[
            pl.BlockSpec((B, BL, D), lambda l: (0, l, 0)),
            pl.BlockSpec((BL, D), lambda l: (l, 0)),
        ],
        out_specs=pl.BlockSpec((B, BL, D), lambda l: (0, l, 0)),
        out_shape=jax.ShapeDtypeStruct((B, L, D), x.dtype),
    )(x, embed_weight[:L])
